# Initial kernel scaffold; baseline (speedup 1.0000x reference)
import functools

import jax
import jax.numpy as jnp
from jax import lax
from jax.experimental import pallas as pl
from jax.experimental.pallas import tpu as pltpu
from jax.experimental.pallas import tpu_sc as plsc

B, S, E, V = 4096, 200, 64, 100000
GROUP = 5
AVOCAB = 52
NC, NS, L = 2, 16, 16
NW = NC * NS
BPW = B // NW
SPAD = 208
NJ = E // L


def _rsqrt16(v):
  i = plsc.bitcast(v, jnp.int32)
  y = plsc.bitcast(
      jnp.full((L,), 0x5F3759DF, jnp.int32) - lax.shift_right_logical(i, 1),
      jnp.float32)
  h = v * 0.5
  for _ in range(3):
    y = y * (1.5 - h * y * y)
  return y


def _body(ids_hbm, word_hbm, pos_hbm, anum_hbm, asset_hbm, g_hbm, be_hbm,
          out_hbm, posasset, asset, anum, gam, bet, ids, rows, outb, sem):
  wid = lax.axis_index("s") * NC + lax.axis_index("c")

  pltpu.sync_copy(pos_hbm.at[pl.ds(0, S)], posasset)
  pltpu.sync_copy(asset_hbm, asset)
  pltpu.sync_copy(anum_hbm, anum)
  pltpu.sync_copy(g_hbm, gam)
  pltpu.sync_copy(be_hbm, bet)
  ids[pl.ds(SPAD - L, L)] = jnp.zeros((L,), jnp.int32)

  @pl.loop(0, S)
  def _(t):
    a = t // GROUP
    for j in range(NJ):
      sl = pl.ds(j * L, L)
      posasset[t, sl] = posasset[t, sl] + asset[a, sl]

  gvec = [gam[pl.ds(j * L, L)] for j in range(NJ)]
  bvec = [bet[pl.ds(j * L, L)] for j in range(NJ)]

  @pl.loop(0, BPW)
  def _(bl):
    b = wid * BPW + bl
    pltpu.sync_copy(ids_hbm.at[b], ids.at[pl.ds(0, S)])

    cnt = jnp.zeros((L,), jnp.int32)
    one = jnp.ones((L,), jnp.int32)
    zero = jnp.zeros((L,), jnp.int32)
    for k in range(SPAD // L):
      cnt = cnt + jnp.where(ids[pl.ds(k * L, L)] != 0, one, zero)
    aidx = jnp.sum(cnt) // GROUP
    avec = [anum[aidx, pl.ds(j * L, L)] for j in range(NJ)]

    d1 = pltpu.async_copy(word_hbm.at[ids.at[pl.ds(0, 96)]],
                          rows.at[pl.ds(0, 96)], sem)
    d2 = pltpu.async_copy(word_hbm.at[ids.at[pl.ds(96, 104)]],
                          rows.at[pl.ds(96, 104)], sem)
    d1.wait()
    d2.wait()

    @pl.loop(0, S)
    def _(t):
      x = [rows[t, pl.ds(j * L, L)] + posasset[t, pl.ds(j * L, L)] + avec[j]
           for j in range(NJ)]
      sv = (x[0] + x[1]) + (x[2] + x[3])
      tot = jnp.sum(sv)
      q = [xj * xj for xj in x]
      qv = (q[0] + q[1]) + (q[2] + q[3])
      tot2 = jnp.sum(qv)
      mean = tot * (1.0 / E)
      var = tot2 * (1.0 / E) - mean * mean
      inv = _rsqrt16(jnp.broadcast_to(var + 1e-5, (L,)))
      for j in range(NJ):
        outb[t, pl.ds(j * L, L)] = (x[j] - mean) * inv * gvec[j] + bvec[j]

    pltpu.sync_copy(outb, out_hbm.at[b])


_mesh = plsc.VectorSubcoreMesh(
    core_axis_name="c", subcore_axis_name="s", num_cores=NC, num_subcores=NS)

_kern = functools.partial(
    pl.kernel,
    out_type=jax.ShapeDtypeStruct((B, S, E), jnp.float32),
    mesh=_mesh,
    scratch_types=[
        pltpu.VMEM((S, E), jnp.float32),
        pltpu.VMEM((AVOCAB, E), jnp.float32),
        pltpu.VMEM((AVOCAB, E), jnp.float32),
        pltpu.VMEM((E,), jnp.float32),
        pltpu.VMEM((E,), jnp.float32),
        pltpu.VMEM((SPAD,), jnp.int32),
        pltpu.VMEM((S, E), jnp.float32),
        pltpu.VMEM((S, E), jnp.float32),
        pltpu.SemaphoreType.DMA,
    ],
)(_body)


@jax.jit
def kernel(input_ids, word_table, pos_table, asset_num_table, asset_table,
           attr_table, ln_gamma, ln_beta):
  del attr_table
  ids = input_ids.astype(jnp.int32)
  return _kern(ids, word_table, pos_table, asset_num_table, asset_table,
               ln_gamma, ln_beta)


# speedup vs baseline: 2.7670x; 2.7670x over previous
import functools

import jax
import jax.numpy as jnp
from jax import lax
from jax.experimental import pallas as pl
from jax.experimental.pallas import tpu as pltpu
from jax.experimental.pallas import tpu_sc as plsc

B, S, E, V = 4096, 200, 64, 100000
GROUP = 5
AVOCAB = 52
NC, NS, L = 2, 16, 16
NW = NC * NS
BPW = B // NW
SPAD = 208
NJ = E // L


def _rsqrt16(v):
  i = plsc.bitcast(v, jnp.int32)
  y = plsc.bitcast(
      jnp.full((L,), 0x5F3759DF, jnp.int32) - lax.shift_right_logical(i, 1),
      jnp.float32)
  h = v * 0.5
  for _ in range(3):
    y = y * (1.5 - h * y * y)
  return y


def _body(ids_hbm, word_hbm, pos_hbm, anum_hbm, asset_hbm, g_hbm, be_hbm,
          out_hbm, posasset, asset, anum, gam, bet, ids, rows, outb, sem):
  wid = lax.axis_index("s") * NC + lax.axis_index("c")

  pltpu.sync_copy(pos_hbm.at[pl.ds(0, S)], posasset)
  pltpu.sync_copy(asset_hbm, asset)
  pltpu.sync_copy(anum_hbm, anum)
  pltpu.sync_copy(g_hbm, gam)
  pltpu.sync_copy(be_hbm, bet)
  ids[pl.ds(SPAD - L, L)] = jnp.zeros((L,), jnp.int32)

  @pl.loop(0, S)
  def _(t):
    a = t // GROUP
    for j in range(NJ):
      sl = pl.ds(j * L, L)
      posasset[t, sl] = posasset[t, sl] + asset[a, sl]

  gvec = [gam[pl.ds(j * L, L)] for j in range(NJ)]
  bvec = [bet[pl.ds(j * L, L)] for j in range(NJ)]

  @pl.loop(0, BPW)
  def _(bl):
    b = wid * BPW + bl
    pltpu.sync_copy(ids_hbm.at[b], ids.at[pl.ds(0, S)])

    cnt = jnp.zeros((L,), jnp.int32)
    one = jnp.ones((L,), jnp.int32)
    zero = jnp.zeros((L,), jnp.int32)
    for k in range(SPAD // L):
      cnt = cnt + jnp.where(ids[pl.ds(k * L, L)] != 0, one, zero)
    aidx = jnp.sum(cnt) // GROUP
    avec = [anum[aidx, pl.ds(j * L, L)] for j in range(NJ)]

    d1 = pltpu.async_copy(word_hbm.at[ids.at[pl.ds(0, 96)]],
                          rows.at[pl.ds(0, 96)], sem)
    d2 = pltpu.async_copy(word_hbm.at[ids.at[pl.ds(96, 104)]],
                          rows.at[pl.ds(96, 104)], sem)
    d1.wait()
    d2.wait()

    @pl.loop(0, S)
    def _(t):
      x = [rows[t, pl.ds(j * L, L)] + posasset[t, pl.ds(j * L, L)] + avec[j]
           for j in range(NJ)]
      sv = (x[0] + x[1]) + (x[2] + x[3])
      tot = jnp.sum(sv)
      q = [xj * xj for xj in x]
      qv = (q[0] + q[1]) + (q[2] + q[3])
      tot2 = jnp.sum(qv)
      mean = tot * (1.0 / E)
      var = tot2 * (1.0 / E) - mean * mean
      inv = _rsqrt16(jnp.broadcast_to(var + 1e-5, (L,)))
      for j in range(NJ):
        outb[t, pl.ds(j * L, L)] = (x[j] - mean) * inv * gvec[j] + bvec[j]

    pltpu.sync_copy(outb, out_hbm.at[b])


_mesh = plsc.VectorSubcoreMesh(
    core_axis_name="c", subcore_axis_name="s", num_cores=NC, num_subcores=NS)

_kern = functools.partial(
    pl.kernel,
    out_type=jax.ShapeDtypeStruct((B, S, E), jnp.float32),
    mesh=_mesh,
    compiler_params=pltpu.CompilerParams(
        needs_layout_passes=False, use_tc_tiling_on_sc=False),
    scratch_types=[
        pltpu.VMEM((S, E), jnp.float32),
        pltpu.VMEM((AVOCAB, E), jnp.float32),
        pltpu.VMEM((AVOCAB, E), jnp.float32),
        pltpu.VMEM((E,), jnp.float32),
        pltpu.VMEM((E,), jnp.float32),
        pltpu.VMEM((SPAD,), jnp.int32),
        pltpu.VMEM((S, E), jnp.float32),
        pltpu.VMEM((S, E), jnp.float32),
        pltpu.SemaphoreType.DMA,
    ],
)(_body)


@jax.jit
def kernel(input_ids, word_table, pos_table, asset_num_table, asset_table,
           attr_table, ln_gamma, ln_beta):
  del attr_table
  ids = input_ids.astype(jnp.int32)
  return _kern(ids, word_table, pos_table, asset_num_table, asset_table,
               ln_gamma, ln_beta)


# speedup vs baseline: 3.6544x; 1.3207x over previous
import functools

import jax
import jax.numpy as jnp
from jax import lax
from jax.experimental import pallas as pl
from jax.experimental.pallas import tpu as pltpu
from jax.experimental.pallas import tpu_sc as plsc

B, S, E, V = 4096, 200, 64, 100000
GROUP = 5
AVOCAB = 52
NC, NS, L = 2, 16, 16
NW = NC * NS
BPW = B // NW
SPAD = 208
NJ = E // L


def _rsqrt16(v):
  i = plsc.bitcast(v, jnp.int32)
  y = plsc.bitcast(
      jnp.full((L,), 0x5F3759DF, jnp.int32) - lax.shift_right_logical(i, 1),
      jnp.float32)
  h = v * 0.5
  for _ in range(3):
    y = y * (1.5 - h * y * y)
  return y


def _body(ids_hbm, word_hbm, pos_hbm, anum_hbm, asset_hbm, g_hbm, be_hbm,
          out_hbm, posasset, asset, anum, gam, bet, ids2, rows2, outb2,
          si0, si1, sg0, sg1, so0, so1):
  wid = lax.axis_index("s") * NC + lax.axis_index("c")
  base = wid * BPW
  sem_i = (si0, si1)
  sem_g = (sg0, sg1)
  sem_o = (so0, so1)

  def ids_copy(r, c):
    return pltpu.make_async_copy(
        ids_hbm.at[base + r], ids2.at[c, pl.ds(0, S)], sem_i[c])

  def gather_copies(c):
    return (
        pltpu.make_async_copy(word_hbm.at[ids2.at[c, pl.ds(0, 96)]],
                              rows2.at[c, pl.ds(0, 96)], sem_g[c]),
        pltpu.make_async_copy(word_hbm.at[ids2.at[c, pl.ds(96, 104)]],
                              rows2.at[c, pl.ds(96, 104)], sem_g[c]),
    )

  def out_copy(r, c):
    return pltpu.make_async_copy(outb2.at[c], out_hbm.at[base + r], sem_o[c])

  pltpu.sync_copy(pos_hbm.at[pl.ds(0, S)], posasset)
  pltpu.sync_copy(asset_hbm, asset)
  pltpu.sync_copy(anum_hbm, anum)
  pltpu.sync_copy(g_hbm, gam)
  pltpu.sync_copy(be_hbm, bet)
  for c in (0, 1):
    ids2[c, pl.ds(SPAD - L, L)] = jnp.zeros((L,), jnp.int32)

  @pl.loop(0, S)
  def _(t):
    a = t // GROUP
    for j in range(NJ):
      sl = pl.ds(j * L, L)
      posasset[t, sl] = posasset[t, sl] + asset[a, sl]

  gvec = [gam[pl.ds(j * L, L)] for j in range(NJ)]
  bvec = [bet[pl.ds(j * L, L)] for j in range(NJ)]

  ids_copy(0, 0).start()
  ids_copy(1, 1).start()
  ids_copy(0, 0).wait()
  for d in gather_copies(0):
    d.start()

  @pl.loop(0, BPW // 2)
  def _(h):
    for c in (0, 1):
      r = 2 * h + c
      o = 1 - c

      @pl.when(r + 1 < BPW)
      def _():
        ids_copy(r + 1, o).wait()
        for d in gather_copies(o):
          d.start()

      for d in gather_copies(c):
        d.wait()

      cnt = jnp.zeros((L,), jnp.int32)
      one = jnp.ones((L,), jnp.int32)
      zero = jnp.zeros((L,), jnp.int32)
      for k in range(SPAD // L):
        cnt = cnt + jnp.where(ids2[c, pl.ds(k * L, L)] != 0, one, zero)
      aidx = jnp.sum(cnt) // GROUP
      avec = [anum[aidx, pl.ds(j * L, L)] for j in range(NJ)]

      @pl.when(r + 2 < BPW)
      def _():
        ids_copy(r + 2, c).start()

      @pl.when(r >= 2)
      def _():
        out_copy(r - 2, c).wait()

      @pl.loop(0, S)
      def _(t):
        x = [rows2[c, t, pl.ds(j * L, L)] + posasset[t, pl.ds(j * L, L)]
             + avec[j] for j in range(NJ)]
        sv = (x[0] + x[1]) + (x[2] + x[3])
        tot = jnp.sum(sv)
        q = [xj * xj for xj in x]
        qv = (q[0] + q[1]) + (q[2] + q[3])
        tot2 = jnp.sum(qv)
        mean = tot * (1.0 / E)
        var = tot2 * (1.0 / E) - mean * mean
        inv = _rsqrt16(jnp.broadcast_to(var + 1e-5, (L,)))
        for j in range(NJ):
          outb2[c, t, pl.ds(j * L, L)] = (x[j] - mean) * inv * gvec[j] + bvec[j]

      out_copy(r, c).start()

  out_copy(BPW - 2, 0).wait()
  out_copy(BPW - 1, 1).wait()


_mesh = plsc.VectorSubcoreMesh(
    core_axis_name="c", subcore_axis_name="s", num_cores=NC, num_subcores=NS)

_kern = functools.partial(
    pl.kernel,
    out_type=jax.ShapeDtypeStruct((B, S, E), jnp.float32),
    mesh=_mesh,
    compiler_params=pltpu.CompilerParams(
        needs_layout_passes=False, use_tc_tiling_on_sc=False),
    scratch_types=[
        pltpu.VMEM((S, E), jnp.float32),
        pltpu.VMEM((AVOCAB, E), jnp.float32),
        pltpu.VMEM((AVOCAB, E), jnp.float32),
        pltpu.VMEM((E,), jnp.float32),
        pltpu.VMEM((E,), jnp.float32),
        pltpu.VMEM((2, SPAD), jnp.int32),
        pltpu.VMEM((2, S, E), jnp.float32),
        pltpu.VMEM((2, S, E), jnp.float32),
        pltpu.SemaphoreType.DMA,
        pltpu.SemaphoreType.DMA,
        pltpu.SemaphoreType.DMA,
        pltpu.SemaphoreType.DMA,
        pltpu.SemaphoreType.DMA,
        pltpu.SemaphoreType.DMA,
    ],
)(_body)


@jax.jit
def kernel(input_ids, word_table, pos_table, asset_num_table, asset_table,
           attr_table, ln_gamma, ln_beta):
  del attr_table
  ids = input_ids.astype(jnp.int32)
  return _kern(ids, word_table, pos_table, asset_num_table, asset_table,
               ln_gamma, ln_beta)
